# baseline (speedup 1.0000x reference)
import functools

import jax
import jax.numpy as jnp
from jax import lax
from jax.experimental import pallas as pl
from jax.experimental.pallas import tpu as pltpu
from jax.experimental.pallas import tpu_sc as plsc

VOCAB = 1000000
D = 64
DP = 128
T, S = 4096, 200
B = T * S
NC = 2
NS = 16
NW = NC * NS
TPW = T // NW
NBUF = 8
AHEAD = 6

_mesh = plsc.VectorSubcoreMesh(core_axis_name="c", subcore_axis_name="s")


@functools.partial(
    pl.kernel,
    out_type=jax.ShapeDtypeStruct((T, S, DP), jnp.float32),
    mesh=_mesh,
    compiler_params=pltpu.CompilerParams(use_tc_tiling_on_sc=False),
    scratch_types=[
        pltpu.VMEM((TPW, S), jnp.int32),
        pltpu.VMEM((NBUF, S, D), jnp.float32),
    ]
    + [pltpu.SemaphoreType.DMA] * (2 * NBUF),
)
def _emb_lookup(table_hbm, idx_hbm, out_hbm, idx_v, rows_v, *sems):
    gsem = sems[:NBUF]
    wsem = sems[NBUF:]
    wid = lax.axis_index("s") * NC + lax.axis_index("c")
    t0 = wid * TPW
    pltpu.sync_copy(idx_hbm.at[pl.ds(t0, TPW)], idx_v)

    def gather(j, b):
        return pltpu.make_async_copy(
            table_hbm.at[idx_v.at[j]], rows_v.at[b], gsem[b])

    def write(j, b):
        return pltpu.make_async_copy(
            rows_v.at[pl.ds(b, 1)],
            out_hbm.at[pl.ds(t0 + j, 1), :, 0:D], wsem[b])

    for j in range(AHEAD):
        gather(j, j % NBUF).start()

    def group(g, carry):
        for u in range(NBUF):
            j = NBUF * g + u
            b = u
            a = j + AHEAD
            ab = (u + AHEAD) % NBUF

            @pl.when(jnp.logical_and(a < TPW, a >= NBUF))
            def _():
                write(0, ab).wait()

            @pl.when(a < TPW)
            def _():
                gather(a, ab).start()

            gather(j, b).wait()
            write(j, b).start()
        return carry

    lax.fori_loop(0, TPW // NBUF, group, 0)
    for u in range(NBUF):
        write(0, u).wait()


def kernel(input_ids, wte):
    idx2 = input_ids.astype(jnp.int32) * 2
    wtep = jnp.pad(wte.T, ((0, DP - D), (0, 0))).T.reshape(2 * VOCAB, D)
    outp = _emb_lookup(wtep, idx2)
    return outp[:, :, :D]
